# baseline (speedup 1.0000x reference)
import functools

import jax
import jax.numpy as jnp
from jax import lax
from jax.experimental import pallas as pl
from jax.experimental.pallas import tpu as pltpu
from jax.experimental.pallas import tpu_sc as plsc

N = 10000
E = 320000
D = 128
EMB = 128

NUM_TILES = 32
CHUNK = 112
NCHUNK = 90
EPT = CHUNK * NCHUNK
NP = 10112
ROWS_PER_TILE = NP // 16


NBUF = 4


def _sc_scatter_body(y_hbm, src_hbm, dst_hbm, zeros_hbm, out_hbm,
                     src_v, dst_v, buf0, buf1, acc, gsem0, gsem1):
    bufs = [buf0, buf1]
    gsems = [gsem0, gsem1]
    c = lax.axis_index("c")
    s = lax.axis_index("s")
    wid = s * 2 + c
    row0 = s * ROWS_PER_TILE

    def gath_start(g, b):
        o = g * CHUNK
        pltpu.make_async_copy(y_hbm.at[src_v.at[pl.ds(o, CHUNK)]],
                              bufs[b], gsems[b]).start()

    def gath_wait(g, b):
        o = g * CHUNK
        pltpu.make_async_copy(y_hbm.at[src_v.at[pl.ds(o, CHUNK)]],
                              bufs[b], gsems[b]).wait()

    pltpu.sync_copy(src_hbm.at[wid], src_v)
    pltpu.sync_copy(dst_hbm.at[wid], dst_v)
    pltpu.sync_copy(zeros_hbm, acc.at[pl.ds(row0, ROWS_PER_TILE)])
    plsc.subcore_barrier()

    gath_start(0, 0)

    def body(g2, carry):
        for b in range(2):
            g = g2 * 2 + b
            gath_wait(g, b)
            gath_start(jnp.where(g + 1 < NCHUNK, g + 1, 0), 1 - b)
            pltpu.sync_copy(bufs[b],
                            acc.at[dst_v.at[pl.ds(g * CHUNK, CHUNK)]],
                            add=True)
        return carry

    lax.fori_loop(0, NCHUNK // 2, body, 0)
    gath_wait(0, 0)

    plsc.subcore_barrier()
    pltpu.sync_copy(acc.at[pl.ds(row0, ROWS_PER_TILE)],
                    out_hbm.at[c].at[pl.ds(row0, ROWS_PER_TILE)])


_sc_scatter = functools.partial(
    pl.kernel,
    out_type=jax.ShapeDtypeStruct((2, NP, EMB), jnp.float32),
    mesh=plsc.VectorSubcoreMesh(core_axis_name="c", subcore_axis_name="s"),
    scratch_types=[
        pltpu.VMEM((EPT,), jnp.int32),
        pltpu.VMEM((EPT,), jnp.int32),
        pltpu.VMEM((CHUNK, EMB), jnp.float32),
        pltpu.VMEM((CHUNK, EMB), jnp.float32),
        pltpu.VMEM_SHARED((NP, EMB), jnp.float32),
        pltpu.SemaphoreType.DMA,
        pltpu.SemaphoreType.DMA,
    ],
)(_sc_scatter_body)


def _init_body(x_ref, w1_ref, bb_ref, w2_ref, h_ref, y_ref):
    h = jnp.dot(x_ref[...], w1_ref[...],
                preferred_element_type=jnp.float32) + bb_ref[...]
    h_ref[...] = h
    y_ref[...] = jnp.dot(jnp.maximum(h, 0.0), w2_ref[...],
                         preferred_element_type=jnp.float32)


def _step_body(h_ref, s_ref, w2_ref, y_ref):
    mu = jnp.maximum(h_ref[...] + s_ref[0, :N] + s_ref[1, :N], 0.0)
    y_ref[...] = jnp.dot(mu, w2_ref[...], preferred_element_type=jnp.float32)


def _colsum_body(h_ref, s_ref, cs_ref):
    mu = jnp.maximum(h_ref[...] + s_ref[0, :N] + s_ref[1, :N], 0.0)
    cs_ref[...] = jnp.sum(mu, axis=0, keepdims=True)


def _final_body(cs1_ref, cs2_ref, wl_ref, bl_ref, sim_ref):
    v1 = jnp.dot(cs1_ref[...], wl_ref[...],
                 preferred_element_type=jnp.float32) + bl_ref[...]
    v2 = jnp.dot(cs2_ref[...], wl_ref[...],
                 preferred_element_type=jnp.float32) + bl_ref[...]
    eps = 1e-8
    n1 = jnp.maximum(jnp.sqrt(jnp.sum(v1 * v1)), eps)
    n2 = jnp.maximum(jnp.sqrt(jnp.sum(v2 * v2)), eps)
    sim_ref[...] = (jnp.sum(v1 * v2) / (n1 * n2)).reshape(1, 1)


_init = pl.pallas_call(
    _init_body,
    out_shape=(jax.ShapeDtypeStruct((N, EMB), jnp.float32),
               jax.ShapeDtypeStruct((N, EMB), jnp.float32)),
)

_step = pl.pallas_call(
    _step_body,
    out_shape=jax.ShapeDtypeStruct((N, EMB), jnp.float32),
)

_colsum = pl.pallas_call(
    _colsum_body,
    out_shape=jax.ShapeDtypeStruct((1, EMB), jnp.float32),
)

_final = pl.pallas_call(
    _final_body,
    out_shape=jax.ShapeDtypeStruct((1, 1), jnp.float32),
)


def kernel(x1, edge_index1, x2, edge_index2, W1, b1, W2, b2, Wl, bl):
    bb = (b1 + b2).reshape(1, EMB)
    blr = bl.reshape(1, EMB)
    zeros = jnp.zeros((ROWS_PER_TILE, EMB), jnp.float32)

    pad = jnp.zeros((NUM_TILES, EPT - E // NUM_TILES), jnp.int32)

    def embed(x, ei):
        src = jnp.concatenate(
            [ei[0].reshape(NUM_TILES, -1), pad], axis=1)
        dst = jnp.concatenate(
            [ei[1].reshape(NUM_TILES, -1), pad + N], axis=1)
        h, y = _init(x, W1, bb, W2)
        for _ in range(3):
            s = _sc_scatter(y, src, dst, zeros)
            y = _step(h, s, W2)
        s = _sc_scatter(y, src, dst, zeros)
        return _colsum(h, s)

    cs1 = embed(x1, edge_index1)
    cs2 = embed(x2, edge_index2)
    return _final(cs1, cs2, Wl, blr).reshape(1)


# speedup vs baseline: 1.4235x; 1.4235x over previous
import functools

import jax
import jax.numpy as jnp
from jax import lax
from jax.experimental import pallas as pl
from jax.experimental.pallas import tpu as pltpu
from jax.experimental.pallas import tpu_sc as plsc

N = 10000
E = 320000
D = 128
EMB = 128

NUM_TILES = 32
CHUNK = 200
NCHUNK = 50
EPT = CHUNK * NCHUNK
NP = 10112
ROWS_PER_TILE = NP // 16


NBUF = 4


def _sc_scatter_body(y_hbm, src_hbm, dst_hbm, zeros_hbm, out_hbm,
                     src_v, dst_v, buf, acc):
    c = lax.axis_index("c")
    s = lax.axis_index("s")
    wid = s * 2 + c
    row0 = s * ROWS_PER_TILE

    pltpu.sync_copy(src_hbm.at[wid], src_v)
    pltpu.sync_copy(dst_hbm.at[wid], dst_v)
    pltpu.sync_copy(zeros_hbm, acc.at[pl.ds(row0, ROWS_PER_TILE)])
    plsc.subcore_barrier()

    def body(g, carry):
        o = g * CHUNK
        pltpu.sync_copy(y_hbm.at[src_v.at[pl.ds(o, CHUNK)]], buf)
        pltpu.sync_copy(buf, acc.at[dst_v.at[pl.ds(o, CHUNK)]], add=True)
        return carry

    lax.fori_loop(0, NCHUNK, body, 0)

    plsc.subcore_barrier()
    pltpu.sync_copy(acc.at[pl.ds(row0, ROWS_PER_TILE)],
                    out_hbm.at[c].at[pl.ds(row0, ROWS_PER_TILE)])


_sc_scatter = functools.partial(
    pl.kernel,
    out_type=jax.ShapeDtypeStruct((2, NP, EMB), jnp.float32),
    mesh=plsc.VectorSubcoreMesh(core_axis_name="c", subcore_axis_name="s"),
    scratch_types=[
        pltpu.VMEM((EPT,), jnp.int32),
        pltpu.VMEM((EPT,), jnp.int32),
        pltpu.VMEM((CHUNK, EMB), jnp.float32),
        pltpu.VMEM_SHARED((NP, EMB), jnp.float32),
    ],
)(_sc_scatter_body)


def _init_body(x_ref, w1_ref, bb_ref, w2_ref, h_ref, y_ref):
    h = jnp.dot(x_ref[...], w1_ref[...],
                preferred_element_type=jnp.float32) + bb_ref[...]
    h_ref[...] = h
    y_ref[...] = jnp.dot(jnp.maximum(h, 0.0), w2_ref[...],
                         preferred_element_type=jnp.float32)


def _step_body(h_ref, s_ref, w2_ref, y_ref):
    mu = jnp.maximum(h_ref[...] + s_ref[0, :N] + s_ref[1, :N], 0.0)
    y_ref[...] = jnp.dot(mu, w2_ref[...], preferred_element_type=jnp.float32)


def _colsum_body(h_ref, s_ref, cs_ref):
    mu = jnp.maximum(h_ref[...] + s_ref[0, :N] + s_ref[1, :N], 0.0)
    cs_ref[...] = jnp.sum(mu, axis=0, keepdims=True)


def _final_body(cs1_ref, cs2_ref, wl_ref, bl_ref, sim_ref):
    v1 = jnp.dot(cs1_ref[...], wl_ref[...],
                 preferred_element_type=jnp.float32) + bl_ref[...]
    v2 = jnp.dot(cs2_ref[...], wl_ref[...],
                 preferred_element_type=jnp.float32) + bl_ref[...]
    eps = 1e-8
    n1 = jnp.maximum(jnp.sqrt(jnp.sum(v1 * v1)), eps)
    n2 = jnp.maximum(jnp.sqrt(jnp.sum(v2 * v2)), eps)
    sim_ref[...] = (jnp.sum(v1 * v2) / (n1 * n2)).reshape(1, 1)


_init = pl.pallas_call(
    _init_body,
    out_shape=(jax.ShapeDtypeStruct((N, EMB), jnp.float32),
               jax.ShapeDtypeStruct((N, EMB), jnp.float32)),
)

_step = pl.pallas_call(
    _step_body,
    out_shape=jax.ShapeDtypeStruct((N, EMB), jnp.float32),
)

_colsum = pl.pallas_call(
    _colsum_body,
    out_shape=jax.ShapeDtypeStruct((1, EMB), jnp.float32),
)

_final = pl.pallas_call(
    _final_body,
    out_shape=jax.ShapeDtypeStruct((1, 1), jnp.float32),
)


def kernel(x1, edge_index1, x2, edge_index2, W1, b1, W2, b2, Wl, bl):
    bb = (b1 + b2).reshape(1, EMB)
    blr = bl.reshape(1, EMB)
    zeros = jnp.zeros((ROWS_PER_TILE, EMB), jnp.float32)

    pad = jnp.zeros((NUM_TILES, EPT - E // NUM_TILES), jnp.int32)

    def embed(x, ei):
        src = jnp.concatenate(
            [ei[0].reshape(NUM_TILES, -1), pad], axis=1)
        dst = jnp.concatenate(
            [ei[1].reshape(NUM_TILES, -1), pad + N], axis=1)
        h, y = _init(x, W1, bb, W2)
        for _ in range(3):
            s = _sc_scatter(y, src, dst, zeros)
            y = _step(h, s, W2)
        s = _sc_scatter(y, src, dst, zeros)
        return _colsum(h, s)

    cs1 = embed(x1, edge_index1)
    cs2 = embed(x2, edge_index2)
    return _final(cs1, cs2, Wl, blr).reshape(1)
